# Initial kernel scaffold; baseline (speedup 1.0000x reference)
import jax
import jax.numpy as jnp
from jax.experimental import pallas as pl


def kernel(word_encs, span_idxs, W, bias):
    raise NotImplementedError("write your pallas kernel here")


# speedup vs baseline: 165.2524x; 165.2524x over previous
import jax
import jax.numpy as jnp
from jax.experimental import pallas as pl
from jax.experimental.pallas import tpu as pltpu

B, T, S, V, DIM = 1024, 200, 50, 1000, 16
BB = 8


def _bow_kernel(enc_ref, lo_ref, hi_ref, w_ref, bias_ref, out_ref):
    enc = enc_ref[...]
    lo = lo_ref[...]
    hi = hi_ref[...]
    pos = jax.lax.broadcasted_iota(jnp.int32, (BB, S, T), 2)
    inspan = ((pos >= lo[:, :, None]) & (pos < hi[:, :, None])).astype(jnp.bfloat16)
    vocab = jax.lax.broadcasted_iota(jnp.int32, (BB, T, V), 2)
    onehot = (vocab == enc[:, :, None]).astype(jnp.bfloat16)
    count = jax.lax.dot_general(
        inspan, onehot,
        dimension_numbers=(((2,), (1,)), ((0,), (0,))),
        preferred_element_type=jnp.float32,
    )
    bow = jnp.minimum(count, 1.0)
    out = jax.lax.dot_general(
        bow.reshape(BB * S, V), w_ref[...],
        dimension_numbers=(((1,), (0,)), ((), ())),
        preferred_element_type=jnp.float32,
    )
    out_ref[...] = out.reshape(BB, S, DIM) + bias_ref[...][None, None, :]


def kernel(word_encs, span_idxs, W, bias):
    lo = span_idxs[:, :, 0].astype(jnp.int32)
    hi = span_idxs[:, :, 1].astype(jnp.int32)
    enc = word_encs.astype(jnp.int32)
    grid = (B // BB,)
    return pl.pallas_call(
        _bow_kernel,
        grid=grid,
        in_specs=[
            pl.BlockSpec((BB, T), lambda g: (g, 0)),
            pl.BlockSpec((BB, S), lambda g: (g, 0)),
            pl.BlockSpec((BB, S), lambda g: (g, 0)),
            pl.BlockSpec((V, DIM), lambda g: (0, 0)),
            pl.BlockSpec((DIM,), lambda g: (0,)),
        ],
        out_specs=pl.BlockSpec((BB, S, DIM), lambda g: (g, 0, 0)),
        out_shape=jax.ShapeDtypeStruct((B, S, DIM), jnp.float32),
    )(enc, lo, hi, W, bias)
